# Initial kernel scaffold; baseline (speedup 1.0000x reference)
import jax
import jax.numpy as jnp
from jax.experimental import pallas as pl


def kernel(student_out, teacher_out, edge_index):
    raise NotImplementedError("write your pallas kernel here")


# speedup vs baseline: 2.4304x; 2.4304x over previous
import functools

import jax
import jax.numpy as jnp
from jax import lax
from jax.experimental import pallas as pl
from jax.experimental.pallas import tpu as pltpu
from jax.experimental.pallas import tpu_sc as plsc

_D = 256
_DD = 2 * _D
_L = 16
_CHUNK = 40
_CPAD = 48


def _normalize_body(s_ref, t_ref, o_ref):
    for ref, col in ((s_ref, 0), (t_ref, _D)):
        x = ref[...] * 2.0
        m = jnp.max(x, axis=1, keepdims=True)
        e = jnp.exp(x - m)
        p = e / jnp.sum(e, axis=1, keepdims=True)
        n = jnp.sqrt(jnp.sum(p * p, axis=1, keepdims=True))
        o_ref[:, col:col + _D] = p / jnp.maximum(n, 1e-12)


def _build_table(student_out, teacher_out):
    n = student_out.shape[0]
    block = 1000
    grid = n // block
    return pl.pallas_call(
        _normalize_body,
        grid=(grid,),
        in_specs=[
            pl.BlockSpec((block, _D), lambda i: (i, 0)),
            pl.BlockSpec((block, _D), lambda i: (i, 0)),
        ],
        out_specs=pl.BlockSpec((block, _DD), lambda i: (i, 0)),
        out_shape=jax.ShapeDtypeStruct((n, _DD), jnp.float32),
    )(student_out, teacher_out)


def _edge_loss_body(num_edges, table, srch, dsth, outh,
                    idx_s, idx_d, rows_s, rows_d, pmat, out_v, sem_s, sem_d):
    info = plsc.get_sparse_core_info()
    nc = info.num_cores
    nw = nc * info.num_subcores
    epw = num_edges // nw
    nchunk = epw // _CHUNK
    wid = lax.axis_index("s") * nc + lax.axis_index("c")
    base = wid * epw

    zeros = jnp.zeros((_L,), jnp.float32)
    iota = lax.iota(jnp.int32, _L)

    def zero_pad(r, _):
        for k in range(_DD // _L):
            rows_s[r, pl.ds(k * _L, _L)] = zeros
            rows_d[r, pl.ds(k * _L, _L)] = zeros
        return 0
    lax.fori_loop(_CHUNK, _CPAD, zero_pad, 0)

    def chunk_body(i, loss_vec):
        off = base + i * _CHUNK
        pltpu.sync_copy(srch.at[pl.ds(off, _CHUNK)], idx_s)
        pltpu.sync_copy(dsth.at[pl.ds(off, _CHUNK)], idx_d)
        cs = pltpu.async_copy(table.at[idx_s], rows_s.at[pl.ds(0, _CHUNK)],
                              sem_s)
        cd = pltpu.async_copy(table.at[idx_d], rows_d.at[pl.ds(0, _CHUNK)],
                              sem_d)
        cs.wait()
        cd.wait()

        def group_body(g, lv):
            def edge_body(e16, _):
                e = g * _L + e16
                acc = zeros
                for k in range(_D // _L):
                    acc = acc + (rows_s[e, pl.ds(k * _L, _L)] *
                                 rows_d[e, pl.ds(k * _L, _L)])
                for k in range(_D // _L, _DD // _L):
                    acc = acc - (rows_s[e, pl.ds(k * _L, _L)] *
                                 rows_d[e, pl.ds(k * _L, _L)])
                pmat[e16, :] = acc
                return 0
            lax.fori_loop(0, _L, edge_body, 0)
            diff = zeros
            for c in range(_L):
                diff = diff + plsc.load_gather(
                    pmat, [iota, jnp.full((_L,), c, jnp.int32)])
            return lv + diff * diff

        return lax.fori_loop(0, _CPAD // _L, group_body, loss_vec)

    loss_vec = lax.fori_loop(0, nchunk, chunk_body, zeros)
    out_v[...] = loss_vec * (1.0 / num_edges)
    pltpu.sync_copy(out_v, outh.at[wid])


def _edge_loss(table, src, dst):
    num_edges = src.shape[0]
    info = plsc.get_sparse_core_info()
    nw = info.num_cores * info.num_subcores
    mesh = plsc.VectorSubcoreMesh(core_axis_name="c", subcore_axis_name="s")
    fn = pl.kernel(
        functools.partial(_edge_loss_body, num_edges),
        out_type=jax.ShapeDtypeStruct((nw, _L), jnp.float32),
        mesh=mesh,
        compiler_params=pltpu.CompilerParams(needs_layout_passes=False),
        scratch_types=[
            pltpu.VMEM((_CHUNK,), jnp.int32),
            pltpu.VMEM((_CHUNK,), jnp.int32),
            pltpu.VMEM((_CPAD, _DD), jnp.float32),
            pltpu.VMEM((_CPAD, _DD), jnp.float32),
            pltpu.VMEM((_L, _L), jnp.float32),
            pltpu.VMEM((_L,), jnp.float32),
            pltpu.SemaphoreType.DMA,
            pltpu.SemaphoreType.DMA,
        ],
    )
    return fn(table, src, dst)


def kernel(student_out, teacher_out, edge_index):
    table = _build_table(student_out, teacher_out)
    partials = _edge_loss(table, edge_index[0], edge_index[1])
    return jnp.sum(partials)


# speedup vs baseline: 5.3749x; 2.2115x over previous
import functools

import jax
import jax.numpy as jnp
from jax import lax
from jax.experimental import pallas as pl
from jax.experimental.pallas import tpu as pltpu
from jax.experimental.pallas import tpu_sc as plsc

_D = 256
_DD = 2 * _D
_L = 16
_CHUNK = 48


def _normalize_body(s_ref, t_ref, o_ref):
    for ref, col in ((s_ref, 0), (t_ref, _D)):
        x = ref[...] * 2.0
        m = jnp.max(x, axis=1, keepdims=True)
        e = jnp.exp(x - m)
        p = e / jnp.sum(e, axis=1, keepdims=True)
        n = jnp.sqrt(jnp.sum(p * p, axis=1, keepdims=True))
        o_ref[:, col:col + _D] = p / jnp.maximum(n, 1e-12)


def _build_table(student_out, teacher_out):
    n = student_out.shape[0]
    block = 1000
    grid = n // block
    return pl.pallas_call(
        _normalize_body,
        grid=(grid,),
        in_specs=[
            pl.BlockSpec((block, _D), lambda i: (i, 0)),
            pl.BlockSpec((block, _D), lambda i: (i, 0)),
        ],
        out_specs=pl.BlockSpec((block, _DD), lambda i: (i, 0)),
        out_shape=jax.ShapeDtypeStruct((n, _DD), jnp.float32),
    )(student_out, teacher_out)


def _edge_loss_body(num_edges, table, srch, dsth, outh,
                    idx_s, idx_d, rows_s0, rows_d0, rows_s1, rows_d1,
                    pmat, out_v, sem_s0, sem_d0, sem_s1, sem_d1):
    info = plsc.get_sparse_core_info()
    nc = info.num_cores
    nw = nc * info.num_subcores
    epw = num_edges // nw
    nfull = epw // _CHUNK
    tail = epw - nfull * _CHUNK
    wid = lax.axis_index("s") * nc + lax.axis_index("c")
    base = wid * epw

    zeros = jnp.zeros((_L,), jnp.float32)
    iota = lax.iota(jnp.int32, _L)
    bufs = ((rows_s0, rows_d0, sem_s0, sem_d0),
            (rows_s1, rows_d1, sem_s1, sem_d1))

    pltpu.sync_copy(srch.at[pl.ds(base, epw)], idx_s)
    pltpu.sync_copy(dsth.at[pl.ds(base, epw)], idx_d)

    def start(c, b, n_rows):
        rs, rd, ss, sd = bufs[b]
        pltpu.async_copy(table.at[idx_s.at[pl.ds(c * _CHUNK, n_rows)]],
                         rs.at[pl.ds(0, n_rows)], ss)
        pltpu.async_copy(table.at[idx_d.at[pl.ds(c * _CHUNK, n_rows)]],
                         rd.at[pl.ds(0, n_rows)], sd)

    def wait(b, n_rows):
        rs, rd, ss, sd = bufs[b]
        pltpu.make_async_copy(table.at[idx_s.at[pl.ds(0, n_rows)]],
                              rs.at[pl.ds(0, n_rows)], ss).wait()
        pltpu.make_async_copy(table.at[idx_d.at[pl.ds(0, n_rows)]],
                              rd.at[pl.ds(0, n_rows)], sd).wait()

    def compute(b, ngroups, lv):
        rs, rd, _, _ = bufs[b]

        def group_body(g, lv):
            def edge_body(e16, _):
                e = g * _L + e16
                acc = zeros
                for k in range(_D // _L):
                    acc = acc + (rs[e, pl.ds(k * _L, _L)] *
                                 rd[e, pl.ds(k * _L, _L)])
                for k in range(_D // _L, _DD // _L):
                    acc = acc - (rs[e, pl.ds(k * _L, _L)] *
                                 rd[e, pl.ds(k * _L, _L)])
                pmat[e16, :] = acc
                return 0
            lax.fori_loop(0, _L, edge_body, 0)
            diff = zeros
            for c in range(_L):
                diff = diff + plsc.load_gather(
                    pmat, [iota, jnp.full((_L,), c, jnp.int32)])
            return lv + diff * diff

        return lax.fori_loop(0, ngroups, group_body, lv)

    start(0, 0, _CHUNK)

    def pair_body(i, lv):
        start(2 * i + 1, 1, _CHUNK)
        wait(0, _CHUNK)
        lv = compute(0, _CHUNK // _L, lv)

        @pl.when(i < nfull // 2 - 1)
        def _():
            start(2 * i + 2, 0, _CHUNK)
        wait(1, _CHUNK)
        lv = compute(1, _CHUNK // _L, lv)
        return lv

    loss_vec = lax.fori_loop(0, nfull // 2, pair_body, zeros)

    start(nfull, 0, tail)
    for r in range(tail, _L):
        for k in range(_DD // _L):
            rows_s0[r, pl.ds(k * _L, _L)] = zeros
            rows_d0[r, pl.ds(k * _L, _L)] = zeros
    wait(0, tail)
    loss_vec = compute(0, 1, loss_vec)

    out_v[...] = loss_vec * (1.0 / num_edges)
    pltpu.sync_copy(out_v, outh.at[wid])


def _edge_loss(table, src, dst):
    num_edges = src.shape[0]
    info = plsc.get_sparse_core_info()
    nw = info.num_cores * info.num_subcores
    epw = num_edges // nw
    mesh = plsc.VectorSubcoreMesh(core_axis_name="c", subcore_axis_name="s")
    fn = pl.kernel(
        functools.partial(_edge_loss_body, num_edges),
        out_type=jax.ShapeDtypeStruct((nw, _L), jnp.float32),
        mesh=mesh,
        compiler_params=pltpu.CompilerParams(needs_layout_passes=False),
        scratch_types=[
            pltpu.VMEM((epw,), jnp.int32),
            pltpu.VMEM((epw,), jnp.int32),
            pltpu.VMEM((_CHUNK, _DD), jnp.float32),
            pltpu.VMEM((_CHUNK, _DD), jnp.float32),
            pltpu.VMEM((_CHUNK, _DD), jnp.float32),
            pltpu.VMEM((_CHUNK, _DD), jnp.float32),
            pltpu.VMEM((_L, _L), jnp.float32),
            pltpu.VMEM((_L,), jnp.float32),
            pltpu.SemaphoreType.DMA,
            pltpu.SemaphoreType.DMA,
            pltpu.SemaphoreType.DMA,
            pltpu.SemaphoreType.DMA,
        ],
    )
    return fn(table, src, dst)


def kernel(student_out, teacher_out, edge_index):
    table = _build_table(student_out, teacher_out)
    partials = _edge_loss(table, edge_index[0], edge_index[1])
    return jnp.sum(partials)
